# baseline (speedup 1.0000x reference)
import functools

import jax
import jax.numpy as jnp
from jax import lax
from jax.experimental import pallas as pl
from jax.experimental.pallas import tpu as pltpu
from jax.experimental.pallas import tpu_sc as plsc

NZ = NX = NY = 96
M = 512
NC = 2
NS = 16
L = 16
NW = NC * NS
ZPW = NZ // NW
NCH = NY // L
CUT2 = 9.0

_GDN = lax.GatherDimensionNumbers(
    offset_dims=(), collapsed_slice_dims=(0,), start_index_map=(0,))


def _lane_gather(v, idx):
    return lax.gather(v, idx, _GDN, (1,),
                      mode=lax.GatherScatterMode.PROMISE_IN_BOUNDS)


def _sc_voxelize(pf_hbm, pi_hbm, zrows_hbm, coords_hbm, vr_hbm, vi_hbm,
                 pf, pi, zrows, cv, axs, exs, ays, eys, accr, acci):
    cid = lax.axis_index("c")
    sid = lax.axis_index("s")
    wid = sid * NC + cid

    pltpu.sync_copy(pf_hbm, pf)
    pltpu.sync_copy(pi_hbm, pi)
    pltpu.sync_copy(zrows_hbm, zrows)
    pltpu.sync_copy(coords_hbm, cv)

    zeros = jnp.zeros((L,), jnp.float32)

    def zero_body(r, carry):
        k = r // NX
        x = r % NX
        for c in range(NCH):
            sl = pl.ds(c * L, L)
            accr[k, x, sl] = zeros
            acci[k, x, sl] = zeros
        return carry

    lax.fori_loop(0, ZPW * NX, zero_body, 0)

    zv = zrows[wid, :]

    def g_body(g, carry):
        piv = pi[g, :]
        z0 = piv[0]
        z1 = piv[1]

        hit = jnp.logical_and(z0 <= wid, wid <= z1)
        for k in range(1, ZPW):
            zk = wid + NW * k
            hit = jnp.logical_or(hit, jnp.logical_and(z0 <= zk, zk <= z1))

        @pl.when(hit)
        def _():
            pfv = pf[g, :]
            cz = pfv[0]
            cx = pfv[1]
            cy = pfv[2]
            isz = pfv[3]
            isx = pfv[4]
            isy = pfv[5]
            rr = pfv[6]
            ri = pfv[7]
            xc0 = piv[6]
            xc1 = piv[7]
            yc0 = piv[4]
            yc1 = piv[5]

            dzv = (zv - cz) * isz
            azv = dzv * dzv
            ezv = jnp.exp(-0.5 * azv)

            for c in range(3):
                xsl = pl.ds((xc0 + c) * L, L)
                ccx = cv[pl.ds((xc0 + c) * L, L)]
                dxv = (ccx - cx) * isx
                axv = dxv * dxv
                axs[xsl] = axv
                exs[xsl] = jnp.exp(-0.5 * axv)
                ysl = pl.ds((yc0 + c) * L, L)
                ccy = cv[pl.ds((yc0 + c) * L, L)]
                dyv = (ccy - cy) * isy
                ayv = dyv * dyv
                ays[ysl] = ayv
                eys[ysl] = jnp.exp(-0.5 * ayv)

            for k in range(ZPW):
                zk = wid + NW * k

                @pl.when(jnp.logical_and(z0 <= zk, zk <= z1))
                def _():
                    azk = azv[k]
                    hr = rr * ezv[k]
                    hi = ri * ezv[k]

                    def yc_body(yc, ycarry):
                        sl = pl.ds(yc * L, L)
                        thr = (CUT2 - azk) - ays[sl]
                        eyv = eys[sl]
                        eyr = eyv * hr
                        eyi = eyv * hi

                        def xc_body(xc, xcarry):
                            xb = xc * L
                            axc = axs[pl.ds(xb, L)]
                            exc = exs[pl.ds(xb, L)]
                            for j in range(L):
                                m = axc[j] <= thr
                                exx = exc[j]
                                tr = jnp.where(m, eyr * exx, 0.0)
                                ti = jnp.where(m, eyi * exx, 0.0)
                                plsc.addupdate(accr.at[k, xb + j, sl], tr)
                                plsc.addupdate(acci.at[k, xb + j, sl], ti)
                            return xcarry

                        lax.fori_loop(xc0, xc1 + 1, xc_body, 0)
                        return ycarry

                    lax.fori_loop(yc0, yc1 + 1, yc_body, 0)

        return carry

    lax.fori_loop(0, M, g_body, 0)

    for k in range(ZPW):
        pltpu.sync_copy(accr.at[pl.ds(k, 1)], vr_hbm.at[pl.ds(wid + NW * k, 1)])
        pltpu.sync_copy(acci.at[pl.ds(k, 1)], vi_hbm.at[pl.ds(wid + NW * k, 1)])


def kernel(centers, scales, rho_real, rho_imag):
    coords = jnp.linspace(-1.0, 1.0, NZ, dtype=jnp.float32)
    step = 2.0 / (NZ - 1)
    eps = jnp.float32(1e-8)

    svec = scales + eps
    inv_s = 1.0 / svec
    rad = 3.0 * svec

    lo = (centers - rad + 1.0) / step
    hi = (centers + rad + 1.0) / step
    i0 = jnp.clip(jnp.floor(lo).astype(jnp.int32), 0, NZ - 1)
    i1 = jnp.clip(jnp.ceil(hi).astype(jnp.int32), 0, NZ - 1)

    zf = jnp.zeros((M,), jnp.float32)
    pf = jnp.stack(
        [centers[:, 0], centers[:, 1], centers[:, 2],
         inv_s[:, 0], inv_s[:, 1], inv_s[:, 2], rho_real, rho_imag,
         zf, zf, zf, zf, zf, zf, zf, zf],
        axis=1)
    zi = jnp.zeros((M,), jnp.int32)
    pi = jnp.stack(
        [i0[:, 0], i1[:, 0], i0[:, 1], i1[:, 1],
         i0[:, 2] // L, i1[:, 2] // L, i0[:, 1] // L, i1[:, 1] // L,
         zi, zi, zi, zi, zi, zi, zi, zi],
        axis=1)

    coords_pad = jnp.concatenate([coords, jnp.zeros((2 * L,), jnp.float32)])
    row_idx = jnp.arange(NW)[:, None] + NW * (jnp.arange(L) % ZPW)[None, :]
    zrows = coords[row_idx]

    mesh = plsc.VectorSubcoreMesh(
        core_axis_name="c", subcore_axis_name="s",
        num_cores=NC, num_subcores=NS)

    vol = jax.ShapeDtypeStruct((NZ, NX, NY), jnp.float32)
    run = functools.partial(
        pl.kernel,
        out_type=(vol, vol),
        mesh=mesh,
        compiler_params=pltpu.CompilerParams(use_tc_tiling_on_sc=False),
        scratch_types=[
            pltpu.VMEM((M, L), jnp.float32),
            pltpu.VMEM((M, L), jnp.int32),
            pltpu.VMEM((NW, L), jnp.float32),
            pltpu.VMEM((8 * L,), jnp.float32),
            pltpu.VMEM((8 * L,), jnp.float32),
            pltpu.VMEM((8 * L,), jnp.float32),
            pltpu.VMEM((8 * L,), jnp.float32),
            pltpu.VMEM((8 * L,), jnp.float32),
            pltpu.VMEM((ZPW, NX, NY), jnp.float32),
            pltpu.VMEM((ZPW, NX, NY), jnp.float32),
        ],
    )(_sc_voxelize)

    vr, vi = run(pf, pi, zrows, coords_pad)
    return lax.complex(vr, vi)


# speedup vs baseline: 1.1526x; 1.1526x over previous
import functools

import jax
import jax.numpy as jnp
from jax import lax
from jax.experimental import pallas as pl
from jax.experimental.pallas import tpu as pltpu
from jax.experimental.pallas import tpu_sc as plsc

NZ = NX = NY = 96
M = 512
NC = 2
NS = 16
L = 16
NW = NC * NS
ZPW = NZ // NW
NCH = NY // L
CUT2 = 9.0

_GDN = lax.GatherDimensionNumbers(
    offset_dims=(), collapsed_slice_dims=(0,), start_index_map=(0,))


def _lane_gather(v, idx):
    return lax.gather(v, idx, _GDN, (1,),
                      mode=lax.GatherScatterMode.PROMISE_IN_BOUNDS)


def _sc_voxelize(pf_hbm, pi_hbm, zrows_hbm, coords_hbm, vr_hbm, vi_hbm,
                 pf, pi, zrows, cv, axs, exs, ays, eys, accr, acci):
    cid = lax.axis_index("c")
    sid = lax.axis_index("s")
    wid = sid * NC + cid

    pltpu.sync_copy(pf_hbm, pf)
    pltpu.sync_copy(pi_hbm, pi)
    pltpu.sync_copy(zrows_hbm, zrows)
    pltpu.sync_copy(coords_hbm, cv)

    zeros = jnp.zeros((L,), jnp.float32)

    def zero_body(r, carry):
        k = r // NX
        x = r % NX
        for c in range(NCH):
            sl = pl.ds(c * L, L)
            accr[k, x, sl] = zeros
            acci[k, x, sl] = zeros
        return carry

    lax.fori_loop(0, ZPW * NX, zero_body, 0)

    zv = zrows[wid, :]

    def g_body(g, carry):
        piv = pi[g, :]
        z0 = piv[0]
        z1 = piv[1]

        hit = jnp.logical_and(z0 <= wid, wid <= z1)
        for k in range(1, ZPW):
            zk = wid + NW * k
            hit = jnp.logical_or(hit, jnp.logical_and(z0 <= zk, zk <= z1))

        @pl.when(hit)
        def _():
            pfv = pf[g, :]
            cz = pfv[0]
            cx = pfv[1]
            cy = pfv[2]
            isz = pfv[3]
            isx = pfv[4]
            isy = pfv[5]
            rr = pfv[6]
            ri = pfv[7]
            xc0 = piv[6]
            xc1 = piv[7]
            yc0 = piv[4]
            yc1 = piv[5]

            dzv = (zv - cz) * isz
            azv = dzv * dzv
            ezv = jnp.exp(-0.5 * azv)

            for c in range(NCH):
                sl = pl.ds(c * L, L)
                cc = cv[pl.ds(c * L, L)]
                dxv = (cc - cx) * isx
                axv = dxv * dxv
                axs[sl] = axv
                exs[sl] = jnp.exp(-0.5 * axv)
                dyv = (cc - cy) * isy
                ayv = dyv * dyv
                ays[sl] = ayv
                eys[sl] = jnp.exp(-0.5 * ayv)

            for k in range(ZPW):
                zk = wid + NW * k

                @pl.when(jnp.logical_and(z0 <= zk, zk <= z1))
                def _():
                    azk = azv[k]
                    hr = rr * ezv[k]
                    hi = ri * ezv[k]

                    def yc_body(yc, ycarry):
                        sl = pl.ds(yc * L, L)
                        thr = (CUT2 - azk) - ays[sl]
                        eyv = eys[sl]
                        eyr = eyv * hr
                        eyi = eyv * hi

                        def xc_body(xc, xcarry):
                            xb = xc * L
                            axc = axs[pl.ds(xb, L)]
                            exc = exs[pl.ds(xb, L)]
                            for j in range(L):
                                m = axc[j] <= thr
                                exx = exc[j]
                                tr = jnp.where(m, eyr * exx, 0.0)
                                ti = jnp.where(m, eyi * exx, 0.0)
                                plsc.addupdate(accr.at[k, xb + j, sl], tr)
                                plsc.addupdate(acci.at[k, xb + j, sl], ti)
                            return xcarry

                        lax.fori_loop(xc0, xc1 + 1, xc_body, 0)
                        return ycarry

                    lax.fori_loop(yc0, yc1 + 1, yc_body, 0)

        return carry

    lax.fori_loop(0, M, g_body, 0)

    for k in range(ZPW):
        pltpu.sync_copy(accr.at[pl.ds(k, 1)], vr_hbm.at[pl.ds(wid + NW * k, 1)])
        pltpu.sync_copy(acci.at[pl.ds(k, 1)], vi_hbm.at[pl.ds(wid + NW * k, 1)])


def kernel(centers, scales, rho_real, rho_imag):
    coords = jnp.linspace(-1.0, 1.0, NZ, dtype=jnp.float32)
    step = 2.0 / (NZ - 1)
    eps = jnp.float32(1e-8)

    svec = scales + eps
    inv_s = 1.0 / svec
    rad = 3.0 * svec

    lo = (centers - rad + 1.0) / step
    hi = (centers + rad + 1.0) / step
    i0 = jnp.clip(jnp.floor(lo).astype(jnp.int32), 0, NZ - 1)
    i1 = jnp.clip(jnp.ceil(hi).astype(jnp.int32), 0, NZ - 1)

    zf = jnp.zeros((M,), jnp.float32)
    pf = jnp.stack(
        [centers[:, 0], centers[:, 1], centers[:, 2],
         inv_s[:, 0], inv_s[:, 1], inv_s[:, 2], rho_real, rho_imag,
         zf, zf, zf, zf, zf, zf, zf, zf],
        axis=1)
    zi = jnp.zeros((M,), jnp.int32)
    pi = jnp.stack(
        [i0[:, 0], i1[:, 0], i0[:, 1], i1[:, 1],
         i0[:, 2] // L, i1[:, 2] // L, i0[:, 1] // L, i1[:, 1] // L,
         zi, zi, zi, zi, zi, zi, zi, zi],
        axis=1)

    coords_pad = jnp.concatenate([coords, jnp.zeros((2 * L,), jnp.float32)])
    row_idx = jnp.arange(NW)[:, None] + NW * (jnp.arange(L) % ZPW)[None, :]
    zrows = coords[row_idx]

    mesh = plsc.VectorSubcoreMesh(
        core_axis_name="c", subcore_axis_name="s",
        num_cores=NC, num_subcores=NS)

    vol = jax.ShapeDtypeStruct((NZ, NX, NY), jnp.float32)
    run = functools.partial(
        pl.kernel,
        out_type=(vol, vol),
        mesh=mesh,
        compiler_params=pltpu.CompilerParams(use_tc_tiling_on_sc=False),
        scratch_types=[
            pltpu.VMEM((M, L), jnp.float32),
            pltpu.VMEM((M, L), jnp.int32),
            pltpu.VMEM((NW, L), jnp.float32),
            pltpu.VMEM((8 * L,), jnp.float32),
            pltpu.VMEM((8 * L,), jnp.float32),
            pltpu.VMEM((8 * L,), jnp.float32),
            pltpu.VMEM((8 * L,), jnp.float32),
            pltpu.VMEM((8 * L,), jnp.float32),
            pltpu.VMEM((ZPW, NX, NY), jnp.float32),
            pltpu.VMEM((ZPW, NX, NY), jnp.float32),
        ],
    )(_sc_voxelize)

    vr, vi = run(pf, pi, zrows, coords_pad)
    return lax.complex(vr, vi)


# speedup vs baseline: 1.1923x; 1.0344x over previous
import functools

import jax
import jax.numpy as jnp
from jax import lax
from jax.experimental import pallas as pl
from jax.experimental.pallas import tpu as pltpu
from jax.experimental.pallas import tpu_sc as plsc

NZ = NX = NY = 96
M = 512
NC = 2
NS = 16
L = 16
NW = NC * NS
ZPW = NZ // NW
NCH = NY // L
CUT2 = 9.0


def _sc_voxelize(pf_hbm, pi_hbm, zrows_hbm, coords_hbm, vr_hbm, vi_hbm,
                 pf, pi, zrows, cv, axs, exs, ays, eys, accr, acci):
    cid = lax.axis_index("c")
    sid = lax.axis_index("s")
    wid = sid * NC + cid

    pltpu.sync_copy(pf_hbm, pf)
    pltpu.sync_copy(pi_hbm, pi)
    pltpu.sync_copy(zrows_hbm, zrows)
    pltpu.sync_copy(coords_hbm, cv)

    zeros = jnp.zeros((L,), jnp.float32)

    def zero_body(r, carry):
        k = r // NX
        x = r % NX
        for c in range(NCH):
            sl = pl.ds(c * L, L)
            accr[k, x, sl] = zeros
            acci[k, x, sl] = zeros
        return carry

    lax.fori_loop(0, ZPW * NX, zero_body, 0)

    zv = zrows[wid, :]

    def g_body(g, carry):
        piv = pi[g, :]
        z0 = piv[0]
        z1 = piv[1]

        hit = jnp.logical_and(z0 <= wid, wid <= z1)
        for k in range(1, ZPW):
            zk = wid + NW * k
            hit = jnp.logical_or(hit, jnp.logical_and(z0 <= zk, zk <= z1))

        @pl.when(hit)
        def _():
            pfv = pf[g, :]
            cz = pfv[0]
            cx = pfv[1]
            cy = pfv[2]
            isz = pfv[3]
            isx = pfv[4]
            isy = pfv[5]
            rr = pfv[6]
            ri = pfv[7]
            x0c = piv[2]
            y0c = piv[3]

            dzv = (zv - cz) * isz
            azv = dzv * dzv
            ezv = jnp.exp(-0.5 * azv)

            for c in range(NCH):
                sl = pl.ds(c * L, L)
                cc = cv[pl.ds(c * L, L)]
                dxv = (cc - cx) * isx
                axv = dxv * dxv
                axs[sl] = axv
                exs[sl] = jnp.exp(-0.5 * axv)
                dyv = (cc - cy) * isy
                ayv = dyv * dyv
                ays[sl] = ayv
                eys[sl] = jnp.exp(-0.5 * ayv)

            for k in range(ZPW):
                zk = wid + NW * k

                @pl.when(jnp.logical_and(z0 <= zk, zk <= z1))
                def _():
                    azk = azv[k]
                    hr = rr * ezv[k]
                    hi = ri * ezv[k]

                    for cy in range(2):
                        sl = pl.ds(y0c + cy * L, L)
                        thr = (CUT2 - azk) - ays[sl]
                        eyv = eys[sl]
                        eyr = eyv * hr
                        eyi = eyv * hi
                        for cx in range(2):
                            xb = x0c + cx * L
                            axc = axs[pl.ds(xb, L)]
                            exc = exs[pl.ds(xb, L)]
                            for j in range(L):
                                m = axc[j] <= thr
                                exx = exc[j]
                                tr = jnp.where(m, eyr * exx, 0.0)
                                ti = jnp.where(m, eyi * exx, 0.0)
                                plsc.addupdate(accr.at[k, xb + j, sl], tr)
                                plsc.addupdate(acci.at[k, xb + j, sl], ti)

        return carry

    lax.fori_loop(0, M, g_body, 0)

    for k in range(ZPW):
        pltpu.sync_copy(accr.at[pl.ds(k, 1)], vr_hbm.at[pl.ds(wid + NW * k, 1)])
        pltpu.sync_copy(acci.at[pl.ds(k, 1)], vi_hbm.at[pl.ds(wid + NW * k, 1)])


def kernel(centers, scales, rho_real, rho_imag):
    coords = jnp.linspace(-1.0, 1.0, NZ, dtype=jnp.float32)
    step = 2.0 / (NZ - 1)
    eps = jnp.float32(1e-8)

    svec = scales + eps
    inv_s = 1.0 / svec
    rad = 3.0 * svec

    lo = (centers - rad + 1.0) / step
    hi = (centers + rad + 1.0) / step
    i0 = jnp.clip(jnp.floor(lo).astype(jnp.int32), 0, NZ - 1)
    i1 = jnp.clip(jnp.ceil(hi).astype(jnp.int32), 0, NZ - 1)

    zf = jnp.zeros((M,), jnp.float32)
    pf = jnp.stack(
        [centers[:, 0], centers[:, 1], centers[:, 2],
         inv_s[:, 0], inv_s[:, 1], inv_s[:, 2], rho_real, rho_imag,
         zf, zf, zf, zf, zf, zf, zf, zf],
        axis=1)
    zi = jnp.zeros((M,), jnp.int32)
    x0c = jnp.minimum(i0[:, 1], NX - 2 * L)
    y0c = jnp.minimum(i0[:, 2], NY - 2 * L)
    pi = jnp.stack(
        [i0[:, 0], i1[:, 0], x0c, y0c,
         zi, zi, zi, zi, zi, zi, zi, zi, zi, zi, zi, zi],
        axis=1)

    coords_pad = jnp.concatenate([coords, jnp.zeros((2 * L,), jnp.float32)])
    row_idx = jnp.arange(NW)[:, None] + NW * (jnp.arange(L) % ZPW)[None, :]
    zrows = coords[row_idx]

    mesh = plsc.VectorSubcoreMesh(
        core_axis_name="c", subcore_axis_name="s",
        num_cores=NC, num_subcores=NS)

    vol = jax.ShapeDtypeStruct((NZ, NX, NY), jnp.float32)
    run = functools.partial(
        pl.kernel,
        out_type=(vol, vol),
        mesh=mesh,
        compiler_params=pltpu.CompilerParams(use_tc_tiling_on_sc=False),
        scratch_types=[
            pltpu.VMEM((M, L), jnp.float32),
            pltpu.VMEM((M, L), jnp.int32),
            pltpu.VMEM((NW, L), jnp.float32),
            pltpu.VMEM((8 * L,), jnp.float32),
            pltpu.VMEM((8 * L,), jnp.float32),
            pltpu.VMEM((8 * L,), jnp.float32),
            pltpu.VMEM((8 * L,), jnp.float32),
            pltpu.VMEM((8 * L,), jnp.float32),
            pltpu.VMEM((ZPW, NX, NY), jnp.float32),
            pltpu.VMEM((ZPW, NX, NY), jnp.float32),
        ],
    )(_sc_voxelize)

    vr, vi = run(pf, pi, zrows, coords_pad)
    return lax.complex(vr, vi)


# speedup vs baseline: 1.1999x; 1.0063x over previous
import functools

import jax
import jax.numpy as jnp
from jax import lax
from jax.experimental import pallas as pl
from jax.experimental.pallas import tpu as pltpu
from jax.experimental.pallas import tpu_sc as plsc

NZ = NX = NY = 96
M = 512
NC = 2
NS = 16
L = 16
NW = NC * NS
ZPW = NZ // NW
NCH = NY // L
CUT2 = 9.0


def _sc_voxelize(pf_hbm, pi_hbm, zrows_hbm, coords_hbm, vr_hbm, vi_hbm,
                 pf, pi, zrows, cv, accr, acci):
    cid = lax.axis_index("c")
    sid = lax.axis_index("s")
    wid = sid * NC + cid

    pltpu.sync_copy(pf_hbm, pf)
    pltpu.sync_copy(pi_hbm, pi)
    pltpu.sync_copy(zrows_hbm, zrows)
    pltpu.sync_copy(coords_hbm, cv)

    zeros = jnp.zeros((L,), jnp.float32)

    def zero_body(r, carry):
        k = r // NX
        x = r % NX
        for c in range(NCH):
            sl = pl.ds(c * L, L)
            accr[k, x, sl] = zeros
            acci[k, x, sl] = zeros
        return carry

    lax.fori_loop(0, ZPW * NX, zero_body, 0)

    zv = zrows[wid, :]

    def g_body(g, carry):
        piv = pi[g, :]
        z0 = piv[0]
        z1 = piv[1]

        hit = jnp.logical_and(z0 <= wid, wid <= z1)
        for k in range(1, ZPW):
            zk = wid + NW * k
            hit = jnp.logical_or(hit, jnp.logical_and(z0 <= zk, zk <= z1))

        @pl.when(hit)
        def _():
            pfv = pf[g, :]
            cz = pfv[0]
            cx = pfv[1]
            cy = pfv[2]
            isz = pfv[3]
            isx = pfv[4]
            isy = pfv[5]
            rr = pfv[6]
            ri = pfv[7]
            x0c = piv[2]
            y0c = piv[3]

            dzv = (zv - cz) * isz
            azv = dzv * dzv
            ezv = jnp.exp(-0.5 * azv)

            axc_l, exc_l, ayc_l, eyc_l = [], [], [], []
            for c in range(2):
                ccx = cv[pl.ds(x0c + c * L, L)]
                dxv = (ccx - cx) * isx
                axv = dxv * dxv
                axc_l.append(axv)
                exc_l.append(jnp.exp(-0.5 * axv))
                ccy = cv[pl.ds(y0c + c * L, L)]
                dyv = (ccy - cy) * isy
                ayv = dyv * dyv
                ayc_l.append(ayv)
                eyc_l.append(jnp.exp(-0.5 * ayv))

            for k in range(ZPW):
                zk = wid + NW * k

                @pl.when(jnp.logical_and(z0 <= zk, zk <= z1))
                def _():
                    azk = azv[k]
                    hr = rr * ezv[k]
                    hi = ri * ezv[k]

                    for cy in range(2):
                        sl = pl.ds(y0c + cy * L, L)
                        thr = (CUT2 - azk) - ayc_l[cy]
                        eyv = eyc_l[cy]
                        eyr = eyv * hr
                        eyi = eyv * hi
                        for cx in range(2):
                            xb = x0c + cx * L
                            axc = axc_l[cx]
                            exc = exc_l[cx]
                            for j in range(L):
                                m = axc[j] <= thr
                                exx = exc[j]
                                tr = jnp.where(m, eyr * exx, 0.0)
                                ti = jnp.where(m, eyi * exx, 0.0)
                                plsc.addupdate(accr.at[k, xb + j, sl], tr)
                                plsc.addupdate(acci.at[k, xb + j, sl], ti)

        return carry

    lax.fori_loop(0, M, g_body, 0)

    for k in range(ZPW):
        pltpu.sync_copy(accr.at[pl.ds(k, 1)], vr_hbm.at[pl.ds(wid + NW * k, 1)])
        pltpu.sync_copy(acci.at[pl.ds(k, 1)], vi_hbm.at[pl.ds(wid + NW * k, 1)])


def kernel(centers, scales, rho_real, rho_imag):
    coords = jnp.linspace(-1.0, 1.0, NZ, dtype=jnp.float32)
    step = 2.0 / (NZ - 1)
    eps = jnp.float32(1e-8)

    svec = scales + eps
    inv_s = 1.0 / svec
    rad = 3.0 * svec

    lo = (centers - rad + 1.0) / step
    hi = (centers + rad + 1.0) / step
    i0 = jnp.clip(jnp.floor(lo).astype(jnp.int32), 0, NZ - 1)
    i1 = jnp.clip(jnp.ceil(hi).astype(jnp.int32), 0, NZ - 1)

    zf = jnp.zeros((M,), jnp.float32)
    pf = jnp.stack(
        [centers[:, 0], centers[:, 1], centers[:, 2],
         inv_s[:, 0], inv_s[:, 1], inv_s[:, 2], rho_real, rho_imag,
         zf, zf, zf, zf, zf, zf, zf, zf],
        axis=1)
    zi = jnp.zeros((M,), jnp.int32)
    x0c = jnp.minimum(i0[:, 1], NX - 2 * L)
    y0c = jnp.minimum(i0[:, 2], NY - 2 * L)
    pi = jnp.stack(
        [i0[:, 0], i1[:, 0], x0c, y0c,
         zi, zi, zi, zi, zi, zi, zi, zi, zi, zi, zi, zi],
        axis=1)

    coords_pad = jnp.concatenate([coords, jnp.zeros((2 * L,), jnp.float32)])
    row_idx = jnp.arange(NW)[:, None] + NW * (jnp.arange(L) % ZPW)[None, :]
    zrows = coords[row_idx]

    mesh = plsc.VectorSubcoreMesh(
        core_axis_name="c", subcore_axis_name="s",
        num_cores=NC, num_subcores=NS)

    vol = jax.ShapeDtypeStruct((NZ, NX, NY), jnp.float32)
    run = functools.partial(
        pl.kernel,
        out_type=(vol, vol),
        mesh=mesh,
        compiler_params=pltpu.CompilerParams(use_tc_tiling_on_sc=False),
        scratch_types=[
            pltpu.VMEM((M, L), jnp.float32),
            pltpu.VMEM((M, L), jnp.int32),
            pltpu.VMEM((NW, L), jnp.float32),
            pltpu.VMEM((8 * L,), jnp.float32),
            pltpu.VMEM((ZPW, NX, NY), jnp.float32),
            pltpu.VMEM((ZPW, NX, NY), jnp.float32),
        ],
    )(_sc_voxelize)

    vr, vi = run(pf, pi, zrows, coords_pad)
    return lax.complex(vr, vi)
